# Initial kernel scaffold; baseline (speedup 1.0000x reference)
import jax
import jax.numpy as jnp
from jax.experimental import pallas as pl


def kernel(indices, values, shape, b):
    raise NotImplementedError("write your pallas kernel here")


# speedup vs baseline: 3.0200x; 3.0200x over previous
import functools

import jax
import jax.numpy as jnp
from jax import lax
from jax.experimental import pallas as pl
from jax.experimental.pallas import tpu as pltpu
from jax.experimental.pallas import tpu_sc as plsc

_NC = 2
_NS = 16
_L = 16
_C = 128


def _spmm_sc(n, d, ch_proc, ch_tot, b, colw, roww, valw):
    rpt = -(-n // (_NS * _C)) * _C
    npad = rpt * _NS
    rblk = _C

    mesh = plsc.VectorSubcoreMesh(core_axis_name="c", subcore_axis_name="s")

    @functools.partial(
        pl.kernel,
        out_type=jax.ShapeDtypeStruct((_NC, npad, d), jnp.float32),
        mesh=mesh,
        scratch_types=[
            pltpu.VMEM((2, _C), jnp.int32),
            pltpu.VMEM((2, _C), jnp.int32),
            pltpu.VMEM((2 * _C,), jnp.float32),
            pltpu.VMEM((_C, d), jnp.float32),
            pltpu.VMEM((_C, d), jnp.float32),
            pltpu.VMEM_SHARED((npad, d), jnp.float32),
            pltpu.SemaphoreType.DMA,
            pltpu.SemaphoreType.DMA,
            pltpu.SemaphoreType.DMA,
            pltpu.SemaphoreType.DMA,
        ],
    )
    def spmm(b_hbm, cols_hbm, rows_hbm, vals_hbm, part_hbm,
             ecols, erows, evals, buf0, buf1, acc, sem0, sem1, semE0, semE1):
        ci = lax.axis_index("c")
        si = lax.axis_index("s")
        wid = ci * _NS + si

        def edge_copies(j, p, sem):
            r = wid * ch_tot + j
            return (
                pltpu.make_async_copy(cols_hbm.at[r], ecols.at[p], sem),
                pltpu.make_async_copy(rows_hbm.at[r], erows.at[p], sem),
                pltpu.make_async_copy(vals_hbm.at[r],
                                      evals.at[pl.ds(p * _C, _C)], sem),
            )

        def issue_edges(j, p, sem):
            for c in edge_copies(j, p, sem):
                c.start()

        def wait_edges(j, p, sem):
            for c in edge_copies(j, p, sem):
                c.wait()

        def gather(j, p, buf, sem):
            return pltpu.make_async_copy(b_hbm.at[ecols.at[p]], buf, sem)

        def _zrow(r, carry):
            for k in range(d // _L):
                buf0[r, pl.ds(k * _L, _L)] = jnp.zeros((_L,), jnp.float32)
            return carry
        lax.fori_loop(0, _C, _zrow, 0)
        for i in range(rpt // rblk):
            pltpu.sync_copy(buf0.at[pl.ds(0, rblk)],
                            acc.at[pl.ds(si * rpt + i * rblk, rblk)])
        plsc.subcore_barrier()

        dnums = lax.GatherDimensionNumbers(
            offset_dims=(), collapsed_slice_dims=(0,), start_index_map=(0,))

        def process(j, p, buf):
            def group(g, carry):
                vvec = evals[pl.ds(p * _C + g * _L, _L)]
                for t in range(_L):
                    vv = lax.gather(
                        vvec, jnp.full((_L, 1), t, jnp.int32), dnums, (1,),
                        mode=lax.GatherScatterMode.PROMISE_IN_BOUNDS)
                    ei = g * _L + t
                    for k in range(d // _L):
                        sl = pl.ds(k * _L, _L)
                        buf[ei, sl] = buf[ei, sl] * vv
                return carry
            lax.fori_loop(0, _C // _L, group, 0)
            pltpu.sync_copy(buf, acc.at[erows.at[p]], add=True)

        issue_edges(0, 0, semE0)
        issue_edges(1, 1, semE1)
        wait_edges(0, 0, semE0)
        gather(0, 0, buf0, sem0).start()

        def step(t, carry):
            j0 = 2 * t
            wait_edges(j0 + 1, 1, semE1)
            gather(j0 + 1, 1, buf1, sem1).start()
            gather(j0, 0, buf0, sem0).wait()
            process(j0, 0, buf0)
            issue_edges(j0 + 2, 0, semE0)
            gather(j0 + 1, 1, buf1, sem1).wait()
            process(j0 + 1, 1, buf1)
            wait_edges(j0 + 2, 0, semE0)
            gather(j0 + 2, 0, buf0, sem0).start()
            issue_edges(j0 + 3, 1, semE1)
            return carry
        lax.fori_loop(0, ch_proc // 2, step, 0)
        gather(ch_proc, 0, buf0, sem0).wait()
        wait_edges(ch_proc + 1, 1, semE1)

        plsc.subcore_barrier()
        for i in range(rpt // rblk):
            sl = pl.ds(si * rpt + i * rblk, rblk)
            pltpu.sync_copy(acc.at[sl], part_hbm.at[ci, sl])

    return spmm(b, colw, roww, valw)


def _sum_partials(partials, d):
    npad = partials.shape[1]
    blk = 512
    assert npad % blk == 0

    def add_body(p_ref, o_ref):
        o_ref[...] = p_ref[0] + p_ref[1]

    return pl.pallas_call(
        add_body,
        grid=(npad // blk,),
        in_specs=[pl.BlockSpec((_NC, blk, d), lambda i: (0, i, 0))],
        out_specs=pl.BlockSpec((blk, d), lambda i: (i, 0)),
        out_shape=jax.ShapeDtypeStruct((npad, d), jnp.float32),
    )(partials)


def kernel(indices, values, shape, b):
    n, d = b.shape
    e = values.shape[0]
    nw = _NC * _NS
    assert e % nw == 0 and d % _L == 0
    ep = e // nw
    ch_proc = -(-ep // _C)
    if ch_proc % 2:
        ch_proc += 1
    ch_tot = ch_proc + 2
    pw = ch_tot * _C

    row = indices[0].astype(jnp.int32)
    col = indices[1].astype(jnp.int32)

    def prep(x, dtype):
        buf = jnp.zeros((nw, pw), dtype)
        return buf.at[:, :ep].set(x.reshape(nw, ep).astype(dtype))

    colw = prep(col, jnp.int32).reshape(nw * ch_tot, _C)
    roww = prep(row, jnp.int32).reshape(nw * ch_tot, _C)
    valw = prep(values, jnp.float32).reshape(nw * ch_tot, _C)

    partials = _spmm_sc(n, d, ch_proc, ch_tot, b, colw, roww, valw)
    return _sum_partials(partials, d)[:n]
